# baseline (speedup 1.0000x reference)
import jax
import jax.numpy as jnp
from jax import lax
from jax.experimental import pallas as pl
from jax.experimental.pallas import tpu as pltpu
from jax.experimental.pallas import tpu_sc as plsc

NC, NS, L = 2, 16, 16
NW = NC * NS
CHUNK = 256
B, S = 16384, 26
N_IDX = B * S
D = 3
N_CHUNKS = N_IDX // CHUNK
CPW = N_CHUNKS // NW
NBUF = 2
GROUPS = CPW // NBUF


def _sc_body(idx_hbm, w1c0, w1c1, w1c2, w2c0, w2c1, w2c2, out_hbm,
             idx_v, g1_v, g2_v, acc_v, sems):
    wid = lax.axis_index("s") * NC + lax.axis_index("c")
    w1cols = (w1c0, w1c1, w1c2)
    w2cols = (w2c0, w2c1, w2c2)

    pltpu.sync_copy(idx_hbm.at[pl.ds(wid * CPW, CPW)], idx_v)

    def start(jj, b):
        for c in range(D):
            pltpu.async_copy(w1cols[c].at[idx_v.at[jj]],
                             g1_v.at[c, b], sems.at[0, b])
            pltpu.async_copy(w2cols[c].at[idx_v.at[jj]],
                             g2_v.at[c, b], sems.at[1, b])

    def drain(jj, b):
        for c in range(D):
            pltpu.make_async_copy(w1cols[c].at[idx_v.at[jj]],
                                  g1_v.at[c, b], sems.at[0, b]).wait()
            pltpu.make_async_copy(w2cols[c].at[idx_v.at[jj]],
                                  g2_v.at[c, b], sems.at[1, b]).wait()

    def compute(j, b):
        base = j * CHUNK
        for c in range(D):
            for m in range(CHUNK // L):
                sl = pl.ds(m * L, L)
                acc_v[c, pl.ds(base + m * L, L)] = (
                    g1_v.at[c, b][sl] + g2_v.at[c, b][sl])

    for b in range(NBUF):
        start(b, b)

    def group(m, carry):
        for b in range(NBUF):
            j = m * NBUF + b
            drain(j, b)
            compute(j, b)
            start(j + NBUF, b)
        return carry

    lax.fori_loop(0, GROUPS - 1, group, 0)

    for b in range(NBUF):
        j = (GROUPS - 1) * NBUF + b
        drain(j, b)
        compute(j, b)

    for c in range(D):
        pltpu.sync_copy(acc_v.at[c],
                        out_hbm.at[c, pl.ds(wid * CPW * CHUNK, CPW * CHUNK)])


def kernel(inputs, W1, W2):
    idx = inputs.reshape(N_CHUNKS, CHUNK).astype(jnp.int32)
    cols = [W[:, c] for W in (W1, W2) for c in range(D)]
    out_planes = pl.kernel(
        _sc_body,
        out_type=jax.ShapeDtypeStruct((D, N_IDX), jnp.float32),
        mesh=plsc.VectorSubcoreMesh(core_axis_name="c", subcore_axis_name="s"),
        compiler_params=pltpu.CompilerParams(
            use_tc_tiling_on_sc=False, needs_layout_passes=False),
        scratch_types=[
            pltpu.VMEM((CPW, CHUNK), jnp.int32),
            pltpu.VMEM((D, NBUF, CHUNK), jnp.float32),
            pltpu.VMEM((D, NBUF, CHUNK), jnp.float32),
            pltpu.VMEM((D, CPW * CHUNK), jnp.float32),
            pltpu.SemaphoreType.DMA((2, NBUF)),
        ],
    )(idx, *cols)
    return out_planes.T.reshape(B, S, D)


# speedup vs baseline: 1.0021x; 1.0021x over previous
import jax
import jax.numpy as jnp
from jax import lax
from jax.experimental import pallas as pl
from jax.experimental.pallas import tpu as pltpu
from jax.experimental.pallas import tpu_sc as plsc

NC, NS, L = 2, 16, 16
NW = NC * NS
CHUNK = 128
B, S = 16384, 26
N_IDX = B * S
D = 3
N_CHUNKS = N_IDX // CHUNK
CPW = N_CHUNKS // NW
NBUF = 4
GROUPS = CPW // NBUF


def _sc_body(idx_hbm, w1c0, w1c1, w1c2, w2c0, w2c1, w2c2, out_hbm,
             idx_v, g1_v, g2_v, acc_v, sems):
    wid = lax.axis_index("s") * NC + lax.axis_index("c")
    w1cols = (w1c0, w1c1, w1c2)
    w2cols = (w2c0, w2c1, w2c2)

    pltpu.sync_copy(idx_hbm.at[pl.ds(wid * CPW, CPW)], idx_v)

    def start(jj, b):
        for c in range(D):
            pltpu.async_copy(w1cols[c].at[idx_v.at[jj]],
                             g1_v.at[c, b], sems.at[0, b])
            pltpu.async_copy(w2cols[c].at[idx_v.at[jj]],
                             g2_v.at[c, b], sems.at[1, b])

    def drain(jj, b):
        for c in range(D):
            pltpu.make_async_copy(w1cols[c].at[idx_v.at[jj]],
                                  g1_v.at[c, b], sems.at[0, b]).wait()
            pltpu.make_async_copy(w2cols[c].at[idx_v.at[jj]],
                                  g2_v.at[c, b], sems.at[1, b]).wait()

    def compute(j, b):
        base = j * CHUNK
        for c in range(D):
            for m in range(CHUNK // L):
                sl = pl.ds(m * L, L)
                acc_v[c, pl.ds(base + m * L, L)] = (
                    g1_v.at[c, b][sl] + g2_v.at[c, b][sl])

    for b in range(NBUF):
        start(b, b)

    def group(m, carry):
        for b in range(NBUF):
            j = m * NBUF + b
            drain(j, b)
            compute(j, b)
            start(j + NBUF, b)
        return carry

    lax.fori_loop(0, GROUPS - 1, group, 0)

    for b in range(NBUF):
        j = (GROUPS - 1) * NBUF + b
        drain(j, b)
        compute(j, b)

    for c in range(D):
        pltpu.sync_copy(acc_v.at[c],
                        out_hbm.at[c, pl.ds(wid * CPW * CHUNK, CPW * CHUNK)])


def kernel(inputs, W1, W2):
    idx = inputs.reshape(N_CHUNKS, CHUNK).astype(jnp.int32)
    cols = [W[:, c] for W in (W1, W2) for c in range(D)]
    out_planes = pl.kernel(
        _sc_body,
        out_type=jax.ShapeDtypeStruct((D, N_IDX), jnp.float32),
        mesh=plsc.VectorSubcoreMesh(core_axis_name="c", subcore_axis_name="s"),
        compiler_params=pltpu.CompilerParams(
            use_tc_tiling_on_sc=False, needs_layout_passes=False),
        scratch_types=[
            pltpu.VMEM((CPW, CHUNK), jnp.int32),
            pltpu.VMEM((D, NBUF, CHUNK), jnp.float32),
            pltpu.VMEM((D, NBUF, CHUNK), jnp.float32),
            pltpu.VMEM((D, CPW * CHUNK), jnp.float32),
            pltpu.SemaphoreType.DMA((2, NBUF)),
        ],
    )(idx, *cols)
    return out_planes.T.reshape(B, S, D)
